# baseline (speedup 1.0000x reference)
import functools

import jax
import jax.numpy as jnp
from jax import lax
from jax.experimental import pallas as pl
from jax.experimental.pallas import tpu as pltpu
from jax.experimental.pallas import tpu_sc as plsc

N = 10000
D = 128
D2 = D // 2
E = 320000
P = 10240
LAM = 0.9
ALP = 1.0 / (LAM + 1.0)
PROP_STEP = 8

NC, NS = 2, 16
NW = NC * NS
K = 128
CHD = 80
CH = 160
EP = NS * CH * K
NBUF = 5
PF = 3
RPT = P // NS
RB = 640

_mesh = plsc.VectorSubcoreMesh(core_axis_name="c", subcore_axis_name="s")


@functools.partial(
    pl.kernel,
    out_type=jax.ShapeDtypeStruct((NC, P), jnp.float32),
    mesh=_mesh,
    compiler_params=pltpu.CompilerParams(use_tc_tiling_on_sc=False),
    scratch_types=[
        pltpu.VMEM((CHD, K), jnp.int32),
        pltpu.VMEM((K,), jnp.float32),
        pltpu.VMEM((RPT,), jnp.float32),
        pltpu.VMEM_SHARED((P,), jnp.float32),
    ],
)
def _deg_kernel(dst_hbm, out_hbm, dst_v, ones_v, zbuf, dacc):
    cid = lax.axis_index("c")
    sid = lax.axis_index("s")
    wid = sid * NC + cid
    pltpu.sync_copy(dst_hbm.at[wid], dst_v)

    def _z(i, c):
        zbuf[pl.ds(i * 16, 16)] = jnp.zeros((16,), jnp.float32)
        return c
    lax.fori_loop(0, RPT // 16, _z, 0)

    def _o(i, c):
        ones_v[pl.ds(i * 16, 16)] = jnp.ones((16,), jnp.float32)
        return c
    lax.fori_loop(0, K // 16, _o, 0)

    pltpu.sync_copy(zbuf, dacc.at[pl.ds(sid * RPT, RPT)])
    plsc.subcore_barrier()

    def _s(j, c):
        pltpu.sync_copy(ones_v, dacc.at[dst_v.at[j]], add=True)
        return c
    lax.fori_loop(0, CHD, _s, 0)

    plsc.subcore_barrier()
    pltpu.sync_copy(dacc.at[pl.ds(sid * RPT, RPT)],
                    out_hbm.at[cid, pl.ds(sid * RPT, RPT)])


@functools.partial(
    pl.kernel,
    out_type=[jax.ShapeDtypeStruct((P, D2), jnp.float32),
              jax.ShapeDtypeStruct((P, D2), jnp.float32)],
    mesh=_mesh,
    compiler_params=pltpu.CompilerParams(use_tc_tiling_on_sc=False),
    scratch_types=[
        pltpu.VMEM((CH, K), jnp.int32),
        pltpu.VMEM((CH, K), jnp.int32),
        pltpu.VMEM((NBUF, K, D2), jnp.float32),
        pltpu.VMEM_SHARED((P, D2), jnp.float32),
        pltpu.SemaphoreType.DMA((NBUF,)),
        pltpu.SemaphoreType.DMA((NBUF,)),
    ],
)
def _scatter_kernel(s0_hbm, s1_hbm, src_hbm, dst_hbm, a0_hbm, a1_hbm,
                    src_v, dst_v, rowbuf, acc, gsem, ssem):
    cid = lax.axis_index("c")
    sid = lax.axis_index("s")
    pltpu.sync_copy(src_hbm.at[sid], src_v)
    pltpu.sync_copy(dst_hbm.at[sid], dst_v)

    def _z(r, c):
        for cc in range(D2 // 16):
            rowbuf[0, r, pl.ds(cc * 16, 16)] = jnp.zeros((16,), jnp.float32)
        return c
    lax.fori_loop(0, K, _z, 0)
    for t in range(RPT // K):
        pltpu.sync_copy(rowbuf.at[0], acc.at[pl.ds(sid * RPT + t * K, K)])
    plsc.subcore_barrier()

    def _ring(s_hbm):
        for b in range(PF):
            pltpu.async_copy(s_hbm.at[src_v.at[b]], rowbuf.at[b], gsem.at[b])

        def _body(it, c):
            j0 = it * NBUF
            for u in range(NBUF):
                j = j0 + u
                b = u % NBUF
                pltpu.make_async_copy(s_hbm.at[src_v.at[j]], rowbuf.at[b],
                                      gsem.at[b]).wait()
                pltpu.async_copy(rowbuf.at[b], acc.at[dst_v.at[j]],
                                 ssem.at[b], add=True)
                bn = (u + PF) % NBUF
                jn = j + PF

                @pl.when(jn < CH)
                def _():
                    @pl.when(jn >= NBUF)
                    def _():
                        pltpu.make_async_copy(rowbuf.at[bn],
                                              acc.at[dst_v.at[jn - NBUF]],
                                              ssem.at[bn]).wait()
                    pltpu.async_copy(s_hbm.at[src_v.at[jn]], rowbuf.at[bn],
                                     gsem.at[bn])
            return c
        lax.fori_loop(0, CH // NBUF, _body, 0)

        for u in range(NBUF):
            j = CH - NBUF + u
            b = j % NBUF
            pltpu.make_async_copy(rowbuf.at[b], acc.at[dst_v.at[j]],
                                  ssem.at[b]).wait()

    @pl.when(cid == 0)
    def _():
        _ring(s0_hbm)

    @pl.when(cid != 0)
    def _():
        _ring(s1_hbm)

    plsc.subcore_barrier()

    @pl.when(cid == 0)
    def _():
        pltpu.sync_copy(acc.at[pl.ds(sid * RPT, RPT)],
                        a0_hbm.at[pl.ds(sid * RPT, RPT)])

    @pl.when(cid != 0)
    def _():
        pltpu.sync_copy(acc.at[pl.ds(sid * RPT, RPT)],
                        a1_hbm.at[pl.ds(sid * RPT, RPT)])


def _mm_body(x_ref, w_ref, b_ref, o_ref, *, relu):
    x = x_ref[...]
    if relu:
        x = jnp.maximum(x, 0.0)
    o_ref[...] = lax.dot_general(x, w_ref[...], (((1,), (1,)), ((), ())),
                                 preferred_element_type=jnp.float32) + b_ref[...]


def _mm(x, w, b, relu):
    return pl.pallas_call(
        functools.partial(_mm_body, relu=relu),
        grid=(P // RB,),
        in_specs=[
            pl.BlockSpec((RB, D), lambda i: (i, 0)),
            pl.BlockSpec((D, D), lambda i: (0, 0)),
            pl.BlockSpec((1, D), lambda i: (0, 0)),
        ],
        out_specs=pl.BlockSpec((RB, D), lambda i: (i, 0)),
        out_shape=jax.ShapeDtypeStruct((P, D), jnp.float32),
    )(x, w, b)


def _prep_body(x_ref, dmbh_ref, dmb1_ref, c_ref, s0_ref, s1_ref):
    x = x_ref[...]
    c_ref[...] = ALP * x * dmb1_ref[...]
    s = x * dmbh_ref[...]
    s0_ref[...] = s[:, :D2]
    s1_ref[...] = s[:, D2:]


def _prep(x, dmbh, dmb1):
    return pl.pallas_call(
        _prep_body,
        grid=(P // RB,),
        in_specs=[
            pl.BlockSpec((RB, D), lambda i: (i, 0)),
            pl.BlockSpec((RB, 1), lambda i: (i, 0)),
            pl.BlockSpec((RB, 1), lambda i: (i, 0)),
        ],
        out_specs=[
            pl.BlockSpec((RB, D), lambda i: (i, 0)),
            pl.BlockSpec((RB, D2), lambda i: (i, 0)),
            pl.BlockSpec((RB, D2), lambda i: (i, 0)),
        ],
        out_shape=[
            jax.ShapeDtypeStruct((P, D), jnp.float32),
            jax.ShapeDtypeStruct((P, D2), jnp.float32),
            jax.ShapeDtypeStruct((P, D2), jnp.float32),
        ],
    )(x, dmbh, dmb1)


def _update_body(y_ref, a0_ref, a1_ref, c_ref, dmbh_ref,
                 yo_ref, s0_ref, s1_ref):
    dm = dmbh_ref[...]
    t = jnp.concatenate([a0_ref[...], a1_ref[...]], axis=1)
    y = (1.0 - ALP) * y_ref[...] + (ALP * LAM) * (t * dm) + c_ref[...]
    yo_ref[...] = y
    s = y * dm
    s0_ref[...] = s[:, :D2]
    s1_ref[...] = s[:, D2:]


def _update(y, a0, a1, c, dmbh):
    return pl.pallas_call(
        _update_body,
        grid=(P // RB,),
        in_specs=[
            pl.BlockSpec((RB, D), lambda i: (i, 0)),
            pl.BlockSpec((RB, D2), lambda i: (i, 0)),
            pl.BlockSpec((RB, D2), lambda i: (i, 0)),
            pl.BlockSpec((RB, D), lambda i: (i, 0)),
            pl.BlockSpec((RB, 1), lambda i: (i, 0)),
        ],
        out_specs=[
            pl.BlockSpec((RB, D), lambda i: (i, 0)),
            pl.BlockSpec((RB, D2), lambda i: (i, 0)),
            pl.BlockSpec((RB, D2), lambda i: (i, 0)),
        ],
        out_shape=[
            jax.ShapeDtypeStruct((P, D), jnp.float32),
            jax.ShapeDtypeStruct((P, D2), jnp.float32),
            jax.ShapeDtypeStruct((P, D2), jnp.float32),
        ],
    )(y, a0, a1, c, dmbh)


def kernel(feat, edge_index, W1, b1, W2, b2):
    src = edge_index[0].astype(jnp.int32)
    dst = edge_index[1].astype(jnp.int32)
    fill = jnp.arange(EP - E, dtype=jnp.int32)
    src_p = jnp.concatenate([src, fill % N])
    dst_p = jnp.concatenate([dst, N + fill % (P - N)])
    src_w = src_p.reshape(NS, CH, K)
    dst_w = dst_p.reshape(NS, CH, K)
    dst_d = dst_p.reshape(NW, CHD, K)
    feat_p = jnp.pad(feat, ((0, P - N), (0, 0)))
    b1r = b1.reshape(1, D)
    b2r = b2.reshape(1, D)

    X = _mm(feat_p, W1, b1r, relu=False)
    deg2 = _deg_kernel(dst_d)
    deg = deg2[0] + deg2[1]
    db = LAM * deg + (1.0 - LAM)
    m = jnp.arange(P) < N
    dmbh = jnp.where(m, lax.rsqrt(db), 0.0)[:, None]
    dmb1 = jnp.where(m, 1.0 / db, 0.0)[:, None]
    C, S0, S1 = _prep(X, dmbh, dmb1)

    Y = X
    for _ in range(PROP_STEP):
        A0, A1 = _scatter_kernel(S0, S1, src_w, dst_w)
        Y, S0, S1 = _update(Y, A0, A1, C, dmbh)

    out = _mm(Y, W2, b2r, relu=True)
    return out[:N]


# speedup vs baseline: 1.1137x; 1.1137x over previous
import functools

import jax
import jax.numpy as jnp
from jax import lax
from jax.experimental import pallas as pl
from jax.experimental.pallas import tpu as pltpu
from jax.experimental.pallas import tpu_sc as plsc

N = 10000
D = 128
D2 = D // 2
E = 320000
P = 10240
LAM = 0.9
ALP = 1.0 / (LAM + 1.0)
PROP_STEP = 8

NC, NS = 2, 16
NW = NC * NS
K = 128
CHD = 80
CH = 160
EP = NS * CH * K
NBUF = 4
PF = 2
RPT = P // NS
BLK = RPT // K
RB = 640

_mesh = plsc.VectorSubcoreMesh(core_axis_name="c", subcore_axis_name="s")


@functools.partial(
    pl.kernel,
    out_type=jax.ShapeDtypeStruct((NC, P), jnp.float32),
    mesh=_mesh,
    compiler_params=pltpu.CompilerParams(use_tc_tiling_on_sc=False),
    scratch_types=[
        pltpu.VMEM((CHD, K), jnp.int32),
        pltpu.VMEM((K,), jnp.float32),
        pltpu.VMEM((RPT,), jnp.float32),
        pltpu.VMEM_SHARED((P,), jnp.float32),
    ],
)
def _deg_kernel(dst_hbm, out_hbm, dst_v, ones_v, zbuf, dacc):
    cid = lax.axis_index("c")
    sid = lax.axis_index("s")
    wid = sid * NC + cid
    pltpu.sync_copy(dst_hbm.at[wid], dst_v)

    def _z(i, c):
        zbuf[pl.ds(i * 16, 16)] = jnp.zeros((16,), jnp.float32)
        return c
    lax.fori_loop(0, RPT // 16, _z, 0)

    def _o(i, c):
        ones_v[pl.ds(i * 16, 16)] = jnp.ones((16,), jnp.float32)
        return c
    lax.fori_loop(0, K // 16, _o, 0)

    pltpu.sync_copy(zbuf, dacc.at[pl.ds(sid * RPT, RPT)])
    plsc.subcore_barrier()

    def _s(j, c):
        pltpu.sync_copy(ones_v, dacc.at[dst_v.at[j]], add=True)
        return c
    lax.fori_loop(0, CHD, _s, 0)

    plsc.subcore_barrier()
    pltpu.sync_copy(dacc.at[pl.ds(sid * RPT, RPT)],
                    out_hbm.at[cid, pl.ds(sid * RPT, RPT)])


@functools.partial(
    pl.kernel,
    out_type=[jax.ShapeDtypeStruct((P, D2), jnp.float32),
              jax.ShapeDtypeStruct((P, D2), jnp.float32),
              jax.ShapeDtypeStruct((P, D2), jnp.float32),
              jax.ShapeDtypeStruct((P, D2), jnp.float32)],
    mesh=_mesh,
    compiler_params=pltpu.CompilerParams(use_tc_tiling_on_sc=False),
    scratch_types=[
        pltpu.VMEM((CH, K), jnp.int32),
        pltpu.VMEM((CH, K), jnp.int32),
        pltpu.VMEM((NBUF + 1, K, D2), jnp.float32),
        pltpu.VMEM((K, 16), jnp.float32),
        pltpu.VMEM_SHARED((P, D2), jnp.float32),
        pltpu.SemaphoreType.DMA((NBUF,)),
        pltpu.SemaphoreType.DMA((NBUF,)),
    ],
)
def _prop_kernel(x0, x1, si0, si1, c0, c1, dmb16_hbm, src_hbm, dst_hbm,
                 y0, y1, s0, s1,
                 src_v, dst_v, rowbuf, dmb_v, acc, gsem, ssem):
    cid = lax.axis_index("c")
    sid = lax.axis_index("s")
    pltpu.sync_copy(src_hbm.at[sid], src_v)
    pltpu.sync_copy(dst_hbm.at[sid], dst_v)

    def _z(r, c):
        for cc in range(D2 // 16):
            rowbuf[NBUF, r, pl.ds(cc * 16, 16)] = jnp.zeros((16,), jnp.float32)
        return c
    lax.fori_loop(0, K, _z, 0)

    def _ring(s_hbm):
        for b in range(NBUF):
            pltpu.async_copy(s_hbm.at[src_v.at[b]], rowbuf.at[b], gsem.at[b])

        def _body(it, c):
            j0 = it * NBUF
            for b in range(NBUF):
                j = j0 + b
                pltpu.make_async_copy(s_hbm.at[src_v.at[j]], rowbuf.at[b],
                                      gsem.at[b]).wait()
                pltpu.async_copy(rowbuf.at[b], acc.at[dst_v.at[j]],
                                 ssem.at[b], add=True)
                pltpu.make_async_copy(rowbuf.at[b], acc.at[dst_v.at[j]],
                                      ssem.at[b]).wait()
                pltpu.async_copy(s_hbm.at[src_v.at[j + NBUF]], rowbuf.at[b],
                                 gsem.at[b])
            return c
        lax.fori_loop(0, CH // NBUF - 1, _body, 0)

        for b in range(NBUF):
            j = CH - NBUF + b
            pltpu.make_async_copy(s_hbm.at[src_v.at[j]], rowbuf.at[b],
                                  gsem.at[b]).wait()
            pltpu.async_copy(rowbuf.at[b], acc.at[dst_v.at[j]],
                             ssem.at[b], add=True)
            pltpu.make_async_copy(rowbuf.at[b], acc.at[dst_v.at[j]],
                                  ssem.at[b]).wait()

    def _half(x, si, c, y, s):
        for t in range(BLK):
            blk = pl.ds(sid * RPT + t * K, K)
            pltpu.sync_copy(x.at[blk], rowbuf.at[0])
            pltpu.sync_copy(rowbuf.at[0], y.at[blk])
            pltpu.sync_copy(si.at[blk], rowbuf.at[1])
            pltpu.sync_copy(rowbuf.at[1], s.at[blk])
            pltpu.sync_copy(rowbuf.at[NBUF], acc.at[blk])
        plsc.subcore_barrier()

        def _step(k, cr):
            _ring(s)
            plsc.subcore_barrier()
            for t in range(BLK):
                blk = pl.ds(sid * RPT + t * K, K)
                pltpu.sync_copy(acc.at[blk], rowbuf.at[0])
                pltpu.sync_copy(y.at[blk], rowbuf.at[1])
                pltpu.sync_copy(c.at[blk], rowbuf.at[2])
                pltpu.sync_copy(dmb16_hbm.at[blk], dmb_v)

                def _row(r, cr2):
                    dm = dmb_v[r]
                    a2 = (ALP * LAM) * dm
                    for cc in range(D2 // 16):
                        sl = pl.ds(cc * 16, 16)
                        yn = ((1.0 - ALP) * rowbuf[1, r, sl]
                              + a2 * rowbuf[0, r, sl] + rowbuf[2, r, sl])
                        rowbuf[1, r, sl] = yn
                        rowbuf[2, r, sl] = yn * dm
                    return cr2
                lax.fori_loop(0, K, _row, 0)
                pltpu.sync_copy(rowbuf.at[1], y.at[blk])
                pltpu.sync_copy(rowbuf.at[2], s.at[blk])
                pltpu.sync_copy(rowbuf.at[NBUF], acc.at[blk])
            plsc.subcore_barrier()
            return cr
        lax.fori_loop(0, PROP_STEP, _step, 0)

    @pl.when(cid == 0)
    def _():
        _half(x0, si0, c0, y0, s0)

    @pl.when(cid != 0)
    def _():
        _half(x1, si1, c1, y1, s1)


def _mm1_body(x_ref, w_ref, b_ref, o_ref):
    o_ref[...] = lax.dot_general(x_ref[...], w_ref[...],
                                 (((1,), (1,)), ((), ())),
                                 preferred_element_type=jnp.float32) + b_ref[...]


def _mm1(x, w, b):
    return pl.pallas_call(
        _mm1_body,
        grid=(P // RB,),
        in_specs=[
            pl.BlockSpec((RB, D), lambda i: (i, 0)),
            pl.BlockSpec((D, D), lambda i: (0, 0)),
            pl.BlockSpec((1, D), lambda i: (0, 0)),
        ],
        out_specs=pl.BlockSpec((RB, D), lambda i: (i, 0)),
        out_shape=jax.ShapeDtypeStruct((P, D), jnp.float32),
    )(x, w, b)


def _mm2_body(y0_ref, y1_ref, w_ref, b_ref, o_ref):
    y = jnp.concatenate([y0_ref[...], y1_ref[...]], axis=1)
    y = jnp.maximum(y, 0.0)
    o_ref[...] = lax.dot_general(y, w_ref[...], (((1,), (1,)), ((), ())),
                                 preferred_element_type=jnp.float32) + b_ref[...]


def _mm2(y0, y1, w, b):
    return pl.pallas_call(
        _mm2_body,
        grid=(P // RB,),
        in_specs=[
            pl.BlockSpec((RB, D2), lambda i: (i, 0)),
            pl.BlockSpec((RB, D2), lambda i: (i, 0)),
            pl.BlockSpec((D, D), lambda i: (0, 0)),
            pl.BlockSpec((1, D), lambda i: (0, 0)),
        ],
        out_specs=pl.BlockSpec((RB, D), lambda i: (i, 0)),
        out_shape=jax.ShapeDtypeStruct((P, D), jnp.float32),
    )(y0, y1, w, b)


def _prep_body(x_ref, dmbh_ref, dmb1_ref,
               x0_ref, x1_ref, s0_ref, s1_ref, c0_ref, c1_ref, dmb16_ref):
    x = x_ref[...]
    dm = dmbh_ref[...]
    c = ALP * x * dmb1_ref[...]
    s = x * dm
    x0_ref[...] = x[:, :D2]
    x1_ref[...] = x[:, D2:]
    s0_ref[...] = s[:, :D2]
    s1_ref[...] = s[:, D2:]
    c0_ref[...] = c[:, :D2]
    c1_ref[...] = c[:, D2:]
    dmb16_ref[...] = jnp.broadcast_to(dm, (RB, 16))


def _prep(x, dmbh, dmb1):
    half = jax.ShapeDtypeStruct((P, D2), jnp.float32)
    return pl.pallas_call(
        _prep_body,
        grid=(P // RB,),
        in_specs=[
            pl.BlockSpec((RB, D), lambda i: (i, 0)),
            pl.BlockSpec((RB, 1), lambda i: (i, 0)),
            pl.BlockSpec((RB, 1), lambda i: (i, 0)),
        ],
        out_specs=[pl.BlockSpec((RB, D2), lambda i: (i, 0))] * 6
        + [pl.BlockSpec((RB, 16), lambda i: (i, 0))],
        out_shape=[half] * 6
        + [jax.ShapeDtypeStruct((P, 16), jnp.float32)],
    )(x, dmbh, dmb1)


def kernel(feat, edge_index, W1, b1, W2, b2):
    src = edge_index[0].astype(jnp.int32)
    dst = edge_index[1].astype(jnp.int32)
    fill = jnp.arange(EP - E, dtype=jnp.int32)
    src_p = jnp.concatenate([src, fill % N])
    dst_p = jnp.concatenate([dst, N + fill % (P - N)])
    src_w = src_p.reshape(NS, CH, K)
    dst_w = dst_p.reshape(NS, CH, K)
    dst_d = dst_p.reshape(NW, CHD, K)
    feat_p = jnp.pad(feat, ((0, P - N), (0, 0)))
    b1r = b1.reshape(1, D)
    b2r = b2.reshape(1, D)

    X = _mm1(feat_p, W1, b1r)
    deg2 = _deg_kernel(dst_d)
    deg = deg2[0] + deg2[1]
    db = LAM * deg + (1.0 - LAM)
    m = jnp.arange(P) < N
    dmbh = jnp.where(m, lax.rsqrt(db), 0.0)
    dmb1 = jnp.where(m, 1.0 / db, 0.0)
    X0, X1, S0, S1, C0, C1, DMB16 = _prep(X, dmbh[:, None], dmb1[:, None])

    Y0, Y1, _, _ = _prop_kernel(X0, X1, S0, S1, C0, C1, DMB16, src_w, dst_w)

    out = _mm2(Y0, Y1, W2, b2r)
    return out[:N]


# speedup vs baseline: 1.1177x; 1.0036x over previous
import functools

import jax
import jax.numpy as jnp
from jax import lax
from jax.experimental import pallas as pl
from jax.experimental.pallas import tpu as pltpu
from jax.experimental.pallas import tpu_sc as plsc

N = 10000
D = 128
D2 = D // 2
E = 320000
P = 10240
LAM = 0.9
ALP = 1.0 / (LAM + 1.0)
PROP_STEP = 8

NC, NS = 2, 16
NW = NC * NS
K = 128
CHD = 80
CH = 160
EP = NS * CH * K
NBUF = 4
PF = 2
RPT = P // NS
BLK = RPT // K
RB = 640

_mesh = plsc.VectorSubcoreMesh(core_axis_name="c", subcore_axis_name="s")


@functools.partial(
    pl.kernel,
    out_type=jax.ShapeDtypeStruct((NC, P), jnp.float32),
    mesh=_mesh,
    compiler_params=pltpu.CompilerParams(use_tc_tiling_on_sc=False),
    scratch_types=[
        pltpu.VMEM((CHD, K), jnp.int32),
        pltpu.VMEM((K,), jnp.float32),
        pltpu.VMEM((RPT,), jnp.float32),
        pltpu.VMEM_SHARED((P,), jnp.float32),
    ],
)
def _deg_kernel(dst_hbm, out_hbm, dst_v, ones_v, zbuf, dacc):
    cid = lax.axis_index("c")
    sid = lax.axis_index("s")
    wid = sid * NC + cid
    pltpu.sync_copy(dst_hbm.at[wid], dst_v)

    def _z(i, c):
        zbuf[pl.ds(i * 16, 16)] = jnp.zeros((16,), jnp.float32)
        return c
    lax.fori_loop(0, RPT // 16, _z, 0)

    def _o(i, c):
        ones_v[pl.ds(i * 16, 16)] = jnp.ones((16,), jnp.float32)
        return c
    lax.fori_loop(0, K // 16, _o, 0)

    pltpu.sync_copy(zbuf, dacc.at[pl.ds(sid * RPT, RPT)])
    plsc.subcore_barrier()

    def _s(j, c):
        pltpu.sync_copy(ones_v, dacc.at[dst_v.at[j]], add=True)
        return c
    lax.fori_loop(0, CHD, _s, 0)

    plsc.subcore_barrier()
    pltpu.sync_copy(dacc.at[pl.ds(sid * RPT, RPT)],
                    out_hbm.at[cid, pl.ds(sid * RPT, RPT)])


@functools.partial(
    pl.kernel,
    out_type=[jax.ShapeDtypeStruct((P, D2), jnp.float32),
              jax.ShapeDtypeStruct((P, D2), jnp.float32),
              jax.ShapeDtypeStruct((P, D2), jnp.float32),
              jax.ShapeDtypeStruct((P, D2), jnp.float32)],
    mesh=_mesh,
    compiler_params=pltpu.CompilerParams(use_tc_tiling_on_sc=False),
    scratch_types=[
        pltpu.VMEM((CH, K), jnp.int32),
        pltpu.VMEM((CH, K), jnp.int32),
        pltpu.VMEM((NBUF + 1, K, D2), jnp.float32),
        pltpu.VMEM((K, 16), jnp.float32),
        pltpu.VMEM_SHARED((P, D2), jnp.float32),
        pltpu.SemaphoreType.DMA((NBUF,)),
        pltpu.SemaphoreType.DMA((NBUF,)),
    ],
)
def _prop_kernel(x0, x1, si0, si1, c0, c1, dmb16_hbm, src_hbm, dst_hbm,
                 y0, y1, s0, s1,
                 src_v, dst_v, rowbuf, dmb_v, acc, gsem, ssem):
    cid = lax.axis_index("c")
    sid = lax.axis_index("s")
    pltpu.sync_copy(src_hbm.at[sid], src_v)
    pltpu.sync_copy(dst_hbm.at[sid], dst_v)

    def _z(r, c):
        for cc in range(D2 // 16):
            rowbuf[NBUF, r, pl.ds(cc * 16, 16)] = jnp.zeros((16,), jnp.float32)
        return c
    lax.fori_loop(0, K, _z, 0)

    def _ring(s_hbm):
        for b in range(NBUF):
            pltpu.async_copy(s_hbm.at[src_v.at[b]], rowbuf.at[b], gsem.at[b])

        def _body(it, c):
            j0 = it * NBUF
            for b in range(NBUF):
                j = j0 + b
                pltpu.make_async_copy(s_hbm.at[src_v.at[j]], rowbuf.at[b],
                                      gsem.at[b]).wait()
                pltpu.async_copy(rowbuf.at[b], acc.at[dst_v.at[j]],
                                 ssem.at[b], add=True)
                pltpu.make_async_copy(rowbuf.at[b], acc.at[dst_v.at[j]],
                                      ssem.at[b]).wait()
                pltpu.async_copy(s_hbm.at[src_v.at[j + NBUF]], rowbuf.at[b],
                                 gsem.at[b])
            return c
        lax.fori_loop(0, CH // NBUF - 1, _body, 0)

        for b in range(NBUF):
            j = CH - NBUF + b
            pltpu.make_async_copy(s_hbm.at[src_v.at[j]], rowbuf.at[b],
                                  gsem.at[b]).wait()
            pltpu.async_copy(rowbuf.at[b], acc.at[dst_v.at[j]],
                             ssem.at[b], add=True)
            pltpu.make_async_copy(rowbuf.at[b], acc.at[dst_v.at[j]],
                                  ssem.at[b]).wait()

    def _half(x, si, c, y, s):
        for t in range(BLK):
            blk = pl.ds(sid * RPT + t * K, K)
            pltpu.sync_copy(x.at[blk], rowbuf.at[0])
            pltpu.sync_copy(rowbuf.at[0], y.at[blk])
            pltpu.sync_copy(si.at[blk], rowbuf.at[1])
            pltpu.sync_copy(rowbuf.at[1], s.at[blk])
            pltpu.sync_copy(rowbuf.at[NBUF], acc.at[blk])
        plsc.subcore_barrier()

        def _step(k, cr):
            _ring(s)
            plsc.subcore_barrier()
            for t in range(BLK):
                blk = pl.ds(sid * RPT + t * K, K)
                pltpu.sync_copy(acc.at[blk], rowbuf.at[0])
                pltpu.sync_copy(y.at[blk], rowbuf.at[1])
                pltpu.sync_copy(c.at[blk], rowbuf.at[2])
                pltpu.sync_copy(dmb16_hbm.at[blk], dmb_v)

                def _row(r, cr2):
                    dm = dmb_v[r]
                    a2 = (ALP * LAM) * dm
                    for cc in range(D2 // 16):
                        sl = pl.ds(cc * 16, 16)
                        yn = ((1.0 - ALP) * rowbuf[1, r, sl]
                              + a2 * rowbuf[0, r, sl] + rowbuf[2, r, sl])
                        rowbuf[1, r, sl] = yn
                        rowbuf[2, r, sl] = yn * dm
                    return cr2
                lax.fori_loop(0, K, _row, 0)
                pltpu.sync_copy(rowbuf.at[1], y.at[blk])
                pltpu.sync_copy(rowbuf.at[2], s.at[blk])
                pltpu.sync_copy(rowbuf.at[NBUF], acc.at[blk])
            plsc.subcore_barrier()
            return cr
        lax.fori_loop(0, PROP_STEP, _step, 0)

    @pl.when(cid == 0)
    def _():
        _half(x0, si0, c0, y0, s0)

    @pl.when(cid != 0)
    def _():
        _half(x1, si1, c1, y1, s1)


def _mm1prep_body(x_ref, w_ref, b_ref, deg_ref,
                  x0_ref, x1_ref, s0_ref, s1_ref, c0_ref, c1_ref, dmb16_ref):
    x = lax.dot_general(x_ref[...], w_ref[...], (((1,), (1,)), ((), ())),
                        preferred_element_type=jnp.float32) + b_ref[...]
    d = deg_ref[0] + deg_ref[1]
    db = LAM * d + (1.0 - LAM)
    rid = (lax.broadcasted_iota(jnp.int32, (RB, 1), 0)
           + pl.program_id(0) * RB)
    msk = rid < N
    dm = jnp.where(msk, lax.rsqrt(db), 0.0)
    dmb1 = jnp.where(msk, 1.0 / db, 0.0)
    c = ALP * x * dmb1
    s = x * dm
    x0_ref[...] = x[:, :D2]
    x1_ref[...] = x[:, D2:]
    s0_ref[...] = s[:, :D2]
    s1_ref[...] = s[:, D2:]
    c0_ref[...] = c[:, :D2]
    c1_ref[...] = c[:, D2:]
    dmb16_ref[...] = jnp.broadcast_to(dm, (RB, 16))


def _mm1prep(x, w, b, deg2):
    half = jax.ShapeDtypeStruct((P, D2), jnp.float32)
    return pl.pallas_call(
        _mm1prep_body,
        grid=(P // RB,),
        in_specs=[
            pl.BlockSpec((RB, D), lambda i: (i, 0)),
            pl.BlockSpec((D, D), lambda i: (0, 0)),
            pl.BlockSpec((1, D), lambda i: (0, 0)),
            pl.BlockSpec((NC, RB, 1), lambda i: (0, i, 0)),
        ],
        out_specs=[pl.BlockSpec((RB, D2), lambda i: (i, 0))] * 6
        + [pl.BlockSpec((RB, 16), lambda i: (i, 0))],
        out_shape=[half] * 6
        + [jax.ShapeDtypeStruct((P, 16), jnp.float32)],
    )(x, w, b, deg2)


def _mm2_body(y0_ref, y1_ref, w_ref, b_ref, o_ref):
    y = jnp.concatenate([y0_ref[...], y1_ref[...]], axis=1)
    y = jnp.maximum(y, 0.0)
    o_ref[...] = lax.dot_general(y, w_ref[...], (((1,), (1,)), ((), ())),
                                 preferred_element_type=jnp.float32) + b_ref[...]


def _mm2(y0, y1, w, b):
    return pl.pallas_call(
        _mm2_body,
        grid=(P // RB,),
        in_specs=[
            pl.BlockSpec((RB, D2), lambda i: (i, 0)),
            pl.BlockSpec((RB, D2), lambda i: (i, 0)),
            pl.BlockSpec((D, D), lambda i: (0, 0)),
            pl.BlockSpec((1, D), lambda i: (0, 0)),
        ],
        out_specs=pl.BlockSpec((RB, D), lambda i: (i, 0)),
        out_shape=jax.ShapeDtypeStruct((P, D), jnp.float32),
    )(y0, y1, w, b)


def kernel(feat, edge_index, W1, b1, W2, b2):
    src = edge_index[0].astype(jnp.int32)
    dst = edge_index[1].astype(jnp.int32)
    fill = jnp.arange(EP - E, dtype=jnp.int32)
    src_p = jnp.concatenate([src, fill % N])
    dst_p = jnp.concatenate([dst, N + fill % (P - N)])
    src_w = src_p.reshape(NS, CH, K)
    dst_w = dst_p.reshape(NS, CH, K)
    dst_d = dst_p.reshape(NW, CHD, K)
    feat_p = jnp.pad(feat, ((0, P - N), (0, 0)))
    b1r = b1.reshape(1, D)
    b2r = b2.reshape(1, D)

    deg2 = _deg_kernel(dst_d)
    X0, X1, S0, S1, C0, C1, DMB16 = _mm1prep(feat_p, W1, b1r,
                                             deg2[:, :, None])

    Y0, Y1, _, _ = _prop_kernel(X0, X1, S0, S1, C0, C1, DMB16, src_w, dst_w)

    out = _mm2(Y0, Y1, W2, b2r)
    return out[:N]


# speedup vs baseline: 1.2252x; 1.0962x over previous
import functools

import jax
import jax.numpy as jnp
from jax import lax
from jax.experimental import pallas as pl
from jax.experimental.pallas import tpu as pltpu
from jax.experimental.pallas import tpu_sc as plsc

N = 10000
D = 128
D2 = D // 2
E = 320000
P = 10240
LAM = 0.9
ALP = 1.0 / (LAM + 1.0)
PROP_STEP = 8

NC, NS = 2, 16
NW = NC * NS
K = 128
CHD = 80
CH = 160
EP = NS * CH * K
NBUF = 4
PF = 2
RPT = P // NS
BLK = RPT // K
RB = 640

_mesh = plsc.VectorSubcoreMesh(core_axis_name="c", subcore_axis_name="s")


@functools.partial(
    pl.kernel,
    out_type=jax.ShapeDtypeStruct((NC, P), jnp.float32),
    mesh=_mesh,
    compiler_params=pltpu.CompilerParams(use_tc_tiling_on_sc=False),
    scratch_types=[
        pltpu.VMEM((CHD, K), jnp.int32),
        pltpu.VMEM((K,), jnp.float32),
        pltpu.VMEM((RPT,), jnp.float32),
        pltpu.VMEM_SHARED((P,), jnp.float32),
    ],
)
def _deg_kernel(dst_hbm, out_hbm, dst_v, ones_v, zbuf, dacc):
    cid = lax.axis_index("c")
    sid = lax.axis_index("s")
    wid = sid * NC + cid
    pltpu.sync_copy(dst_hbm.at[wid], dst_v)

    def _z(i, c):
        zbuf[pl.ds(i * 16, 16)] = jnp.zeros((16,), jnp.float32)
        return c
    lax.fori_loop(0, RPT // 16, _z, 0)

    def _o(i, c):
        ones_v[pl.ds(i * 16, 16)] = jnp.ones((16,), jnp.float32)
        return c
    lax.fori_loop(0, K // 16, _o, 0)

    pltpu.sync_copy(zbuf, dacc.at[pl.ds(sid * RPT, RPT)])
    plsc.subcore_barrier()

    def _s(j, c):
        pltpu.sync_copy(ones_v, dacc.at[dst_v.at[j]], add=True)
        return c
    lax.fori_loop(0, CHD, _s, 0)

    plsc.subcore_barrier()
    pltpu.sync_copy(dacc.at[pl.ds(sid * RPT, RPT)],
                    out_hbm.at[cid, pl.ds(sid * RPT, RPT)])


@functools.partial(
    pl.kernel,
    out_type=[jax.ShapeDtypeStruct((P, D2), jnp.float32),
              jax.ShapeDtypeStruct((P, D2), jnp.float32)],
    mesh=_mesh,
    compiler_params=pltpu.CompilerParams(use_tc_tiling_on_sc=False),
    scratch_types=[
        pltpu.VMEM((CH, K), jnp.int32),
        pltpu.VMEM((CH, K), jnp.int32),
        pltpu.VMEM((NBUF + 1, K, D2), jnp.float32),
        pltpu.VMEM((K // 4, D2), jnp.float32),
        pltpu.VMEM((K, 32), jnp.float32),
        pltpu.VMEM_SHARED((P, D2), jnp.float32),
        pltpu.SemaphoreType.DMA((NBUF,)),
        pltpu.SemaphoreType.DMA((NBUF,)),
    ],
)
def _prop_kernel(si0, si1, c0, c1, dmc_hbm, src_hbm, dst_hbm,
                 s0, s1,
                 src_v, dst_v, rowbuf, zbuf, dmcv, acc, gsem, ssem):
    cid = lax.axis_index("c")
    sid = lax.axis_index("s")
    pltpu.sync_copy(src_hbm.at[sid], src_v)
    pltpu.sync_copy(dst_hbm.at[sid], dst_v)

    def _z(r, c):
        for cc in range(D2 // 16):
            zbuf[r, pl.ds(cc * 16, 16)] = jnp.zeros((16,), jnp.float32)
        return c
    lax.fori_loop(0, K // 4, _z, 0)

    def _ring(s_hbm):
        for b in range(NBUF):
            pltpu.async_copy(s_hbm.at[src_v.at[b]], rowbuf.at[b], gsem.at[b])

        def _body(it, c):
            j0 = it * NBUF
            for b in range(NBUF):
                j = j0 + b
                pltpu.make_async_copy(s_hbm.at[src_v.at[j]], rowbuf.at[b],
                                      gsem.at[b]).wait()
                pltpu.async_copy(rowbuf.at[b], acc.at[dst_v.at[j]],
                                 ssem.at[b], add=True)
                pltpu.make_async_copy(rowbuf.at[b], acc.at[dst_v.at[j]],
                                      ssem.at[b]).wait()
                pltpu.async_copy(s_hbm.at[src_v.at[j + NBUF]], rowbuf.at[b],
                                 gsem.at[b])
            return c
        lax.fori_loop(0, CH // NBUF - 1, _body, 0)

        for b in range(NBUF):
            j = CH - NBUF + b
            pltpu.make_async_copy(s_hbm.at[src_v.at[j]], rowbuf.at[b],
                                  gsem.at[b]).wait()
            pltpu.async_copy(rowbuf.at[b], acc.at[dst_v.at[j]],
                             ssem.at[b], add=True)
            pltpu.make_async_copy(rowbuf.at[b], acc.at[dst_v.at[j]],
                                  ssem.at[b]).wait()

    ZR = K // 4

    def _blk(t):
        return pl.ds(sid * RPT + t * K, K)

    def _zero_acc(t, sem):
        base = sid * RPT + t * K
        for z in range(K // ZR):
            pltpu.async_copy(zbuf, acc.at[pl.ds(base + z * ZR, ZR)], sem)

    def _zero_acc_wait(t, sem):
        base = sid * RPT + t * K
        for z in range(K // ZR):
            pltpu.make_async_copy(zbuf, acc.at[pl.ds(base + z * ZR, ZR)],
                                  sem).wait()

    def _half(si, c, s):
        for t in range(BLK):
            pltpu.sync_copy(si.at[_blk(t)], rowbuf.at[0])
            pltpu.sync_copy(rowbuf.at[0], s.at[_blk(t)])
            _zero_acc(t, ssem.at[0])
            _zero_acc_wait(t, ssem.at[0])
        plsc.subcore_barrier()

        def _step(k, cr):
            _ring(s)
            plsc.subcore_barrier()
            pltpu.async_copy(acc.at[_blk(0)], rowbuf.at[0], gsem.at[0])
            pltpu.async_copy(c.at[_blk(0)], rowbuf.at[2], gsem.at[2])
            for t in range(BLK):
                sA = t % 2
                sC = 2 + t % 2
                if t + 1 < BLK:
                    pltpu.async_copy(acc.at[_blk(t + 1)],
                                     rowbuf.at[(t + 1) % 2],
                                     gsem.at[(t + 1) % 2])
                    pltpu.async_copy(c.at[_blk(t + 1)],
                                     rowbuf.at[2 + (t + 1) % 2],
                                     gsem.at[2 + (t + 1) % 2])
                pltpu.sync_copy(s.at[_blk(t)], rowbuf.at[NBUF])
                pltpu.sync_copy(dmc_hbm.at[_blk(t)], dmcv)
                pltpu.make_async_copy(acc.at[_blk(t)], rowbuf.at[sA],
                                      gsem.at[sA]).wait()
                pltpu.make_async_copy(c.at[_blk(t)], rowbuf.at[sC],
                                      gsem.at[sC]).wait()

                def _row(r, cr2):
                    dm = dmcv[r, pl.ds(0, 16)]
                    a1v = (1.0 - ALP) * dmcv[r, pl.ds(16, 16)]
                    a2 = (ALP * LAM) * dm
                    for cc in range(D2 // 16):
                        sl = pl.ds(cc * 16, 16)
                        yn = (a1v * rowbuf[NBUF, r, sl]
                              + a2 * rowbuf[sA, r, sl] + rowbuf[sC, r, sl])
                        rowbuf[NBUF, r, sl] = yn * dm
                    return cr2
                lax.fori_loop(0, K, _row, 0)

                if t >= 2:
                    _zero_acc_wait(t - 2, ssem.at[sC])
                pltpu.sync_copy(rowbuf.at[NBUF], s.at[_blk(t)])
                _zero_acc(t, ssem.at[sC])
            _zero_acc_wait(BLK - 2, ssem.at[2 + (BLK - 2) % 2])
            _zero_acc_wait(BLK - 1, ssem.at[2 + (BLK - 1) % 2])
            plsc.subcore_barrier()
            return cr
        lax.fori_loop(0, PROP_STEP, _step, 0)

    @pl.when(cid == 0)
    def _():
        _half(si0, c0, s0)

    @pl.when(cid != 0)
    def _():
        _half(si1, c1, s1)


def _mm1prep_body(x_ref, w_ref, b_ref, deg_ref,
                  s0_ref, s1_ref, c0_ref, c1_ref, dmc_ref, dminv_ref):
    x = lax.dot_general(x_ref[...], w_ref[...], (((1,), (1,)), ((), ())),
                        preferred_element_type=jnp.float32) + b_ref[...]
    d = deg_ref[0] + deg_ref[1]
    db = LAM * d + (1.0 - LAM)
    rid = (lax.broadcasted_iota(jnp.int32, (RB, 1), 0)
           + pl.program_id(0) * RB)
    msk = rid < N
    dm = jnp.where(msk, lax.rsqrt(db), 0.0)
    dminv = jnp.where(msk, db * dm, 0.0)
    dmb1 = jnp.where(msk, 1.0 / db, 0.0)
    c = ALP * x * dmb1
    s = x * dm
    s0_ref[...] = s[:, :D2]
    s1_ref[...] = s[:, D2:]
    c0_ref[...] = c[:, :D2]
    c1_ref[...] = c[:, D2:]
    dmc_ref[...] = jnp.concatenate(
        [jnp.broadcast_to(dm, (RB, 16)), jnp.broadcast_to(dminv, (RB, 16))],
        axis=1)
    dminv_ref[...] = dminv


def _mm1prep(x, w, b, deg2):
    half = jax.ShapeDtypeStruct((P, D2), jnp.float32)
    return pl.pallas_call(
        _mm1prep_body,
        grid=(P // RB,),
        in_specs=[
            pl.BlockSpec((RB, D), lambda i: (i, 0)),
            pl.BlockSpec((D, D), lambda i: (0, 0)),
            pl.BlockSpec((1, D), lambda i: (0, 0)),
            pl.BlockSpec((NC, RB, 1), lambda i: (0, i, 0)),
        ],
        out_specs=[pl.BlockSpec((RB, D2), lambda i: (i, 0))] * 4
        + [pl.BlockSpec((RB, 32), lambda i: (i, 0)),
           pl.BlockSpec((RB, 1), lambda i: (i, 0))],
        out_shape=[half] * 4
        + [jax.ShapeDtypeStruct((P, 32), jnp.float32),
           jax.ShapeDtypeStruct((P, 1), jnp.float32)],
    )(x, w, b, deg2)


def _mm2_body(s0_ref, s1_ref, dminv_ref, w_ref, b_ref, o_ref):
    y = jnp.concatenate([s0_ref[...], s1_ref[...]], axis=1) * dminv_ref[...]
    y = jnp.maximum(y, 0.0)
    o_ref[...] = lax.dot_general(y, w_ref[...], (((1,), (1,)), ((), ())),
                                 preferred_element_type=jnp.float32) + b_ref[...]


def _mm2(s0, s1, dminv, w, b):
    return pl.pallas_call(
        _mm2_body,
        grid=(P // RB,),
        in_specs=[
            pl.BlockSpec((RB, D2), lambda i: (i, 0)),
            pl.BlockSpec((RB, D2), lambda i: (i, 0)),
            pl.BlockSpec((RB, 1), lambda i: (i, 0)),
            pl.BlockSpec((D, D), lambda i: (0, 0)),
            pl.BlockSpec((1, D), lambda i: (0, 0)),
        ],
        out_specs=pl.BlockSpec((RB, D), lambda i: (i, 0)),
        out_shape=jax.ShapeDtypeStruct((P, D), jnp.float32),
    )(s0, s1, dminv, w, b)


def kernel(feat, edge_index, W1, b1, W2, b2):
    src = edge_index[0].astype(jnp.int32)
    dst = edge_index[1].astype(jnp.int32)
    fill = jnp.arange(EP - E, dtype=jnp.int32)
    src_p = jnp.concatenate([src, fill % N])
    dst_p = jnp.concatenate([dst, N + fill % (P - N)])
    src_w = src_p.reshape(NS, CH, K)
    dst_w = dst_p.reshape(NS, CH, K)
    dst_d = dst_p.reshape(NW, CHD, K)
    feat_p = jnp.pad(feat, ((0, P - N), (0, 0)))
    b1r = b1.reshape(1, D)
    b2r = b2.reshape(1, D)

    deg2 = _deg_kernel(dst_d)
    S0, S1, C0, C1, DMC, DMINV = _mm1prep(feat_p, W1, b1r, deg2[:, :, None])

    SF0, SF1 = _prop_kernel(S0, S1, C0, C1, DMC, src_w, dst_w)

    out = _mm2(SF0, SF1, DMINV, W2, b2r)
    return out[:N]
